# baseline (speedup 1.0000x reference)
import functools

import jax
import jax.numpy as jnp
from jax import lax
from jax.experimental import pallas as pl
from jax.experimental.pallas import tpu as pltpu
from jax.experimental.pallas import tpu_sc as plsc

_NUM_RULES = 1000
_NODES = 8
_EDGES = 12
_D = 64
_LAYERS = 3
_NSYM = 50
_PLANES = _EDGES + _NODES
_TABLE_ROWS = 1024


def _split3(x):
    f32 = jnp.float32
    hi = x.astype(jnp.bfloat16).astype(f32)
    r = x - hi
    mid = r.astype(jnp.bfloat16).astype(f32)
    return hi, mid, r - mid


def _table_body(esym, nsym, en, aemb, bemb, wl, bl, wot, bo, out,
                ohs, h, agg, tab):
    R = _NUM_RULES
    f32 = jnp.float32
    dn = (((1,), (0,)), ((), ()))
    dnt = (((1,), (1,)), ((), ()))

    tab[:, :] = jnp.zeros((2 * _D, _D), f32)
    tab[0:_NSYM, :] = aemb[:, :]
    tab[pl.ds(_D, _NSYM), :] = bemb[:, :]
    iota = lax.broadcasted_iota(jnp.int32, (R, _D), 1)

    def onehot_embed(nplanes, table):
        p1, p2, p3 = _split3(table)
        o = ohs[pl.ds(0, nplanes * R), :]
        return (lax.dot_general(o, p1, dn, preferred_element_type=f32)
                + lax.dot_general(o, p2, dn, preferred_element_type=f32)
                + lax.dot_general(o, p3, dn, preferred_element_type=f32))

    for e in range(_EDGES):
        ohs[pl.ds(e * R, R), :] = (esym[:, e:e + 1] == iota).astype(f32)
    h[pl.ds(0, _EDGES * R), :] = onehot_embed(_EDGES, tab[0:_D, :])
    for n in range(_NODES):
        ohs[pl.ds(n * R, R), :] = (nsym[:, n:n + 1] == iota).astype(f32)
    h[pl.ds(_EDGES * R, _NODES * R), :] = onehot_embed(_NODES, tab[pl.ds(_D, _D), :])

    def nplane(i):
        return pl.ds(pl.multiple_of((_EDGES + i) * R, 8), R)

    node0 = _EDGES * R
    ends = [(en[e, 0], en[e, 1]) for e in range(_EDGES)]
    v = None
    for l in range(_LAYERS):
        last = l == _LAYERS - 1
        if not last:
            for e in range(_EDGES):
                a, b = ends[e]
                agg[pl.ds(e * R, R), :] = (
                    h[pl.ds(e * R, R), :] + h[nplane(a), :] + h[nplane(b), :]
                )
        agg[pl.ds(node0, _NODES * R), :] = h[pl.ds(node0, _NODES * R), :]
        for e in range(_EDGES):
            for k in range(2):
                i = ends[e][k]
                agg[nplane(i), :] = agg[nplane(i), :] + h[pl.ds(e * R, R), :]
        if last:
            v = agg[pl.ds(node0 + (_NODES - 1) * R, R), :]
        else:
            h[:, :] = jnp.maximum(
                lax.dot_general(agg[:, :], wl[l], dnt,
                                preferred_element_type=f32) + bl[l:l + 1, :],
                0.0)
    out[0:R, :] = jnp.tanh(
        lax.dot_general(v, wot[:, :], dnt, preferred_element_type=f32) + bo[:, :])
    out[pl.ds(R, _TABLE_ROWS - R), :] = jnp.zeros((_TABLE_ROWS - R, _D), f32)


def _build_table(esym, nsym, en, aemb, bemb, wl, bl, wot, bo):
    R = _NUM_RULES
    vmem = pl.BlockSpec(memory_space=pltpu.VMEM)
    return pl.pallas_call(
        _table_body,
        out_shape=jax.ShapeDtypeStruct((_TABLE_ROWS, _D), jnp.float32),
        in_specs=[
            vmem,
            vmem,
            pl.BlockSpec(memory_space=pltpu.SMEM),
            vmem,
            vmem,
            vmem,
            vmem,
            vmem,
            vmem,
        ],
        out_specs=pl.BlockSpec(memory_space=pltpu.VMEM),
        scratch_shapes=[
            pltpu.VMEM((_EDGES * R, _D), jnp.float32),
            pltpu.VMEM((_PLANES * R, _D), jnp.float32),
            pltpu.VMEM((_PLANES * R, _D), jnp.float32),
            pltpu.VMEM((2 * _D, _D), jnp.float32),
        ],
    )(esym, nsym, en, aemb, bemb, wl, bl, wot, bo)


@functools.lru_cache(maxsize=None)
def _gather_call(batch):
    info = plsc.get_sparse_core_info()
    nc, ns = info.num_cores, info.num_subcores
    nw = nc * ns
    bpw = batch // nw
    ch = 128
    nchunk = -(-bpw // ch)
    bpad = nchunk * ch
    ga = nchunk // 2
    mesh = plsc.VectorSubcoreMesh(core_axis_name="c", subcore_axis_name="s")

    @functools.partial(
        pl.kernel,
        mesh=mesh,
        compiler_params=pltpu.CompilerParams(use_tc_tiling_on_sc=False),
        out_type=jax.ShapeDtypeStruct((batch, _D), jnp.float32),
        scratch_types=[
            pltpu.VMEM((nchunk, ch), jnp.int32),
            pltpu.VMEM((bpad, _D), jnp.float32),
            pltpu.SemaphoreType.DMA,
            pltpu.SemaphoreType.DMA,
            pltpu.SemaphoreType.DMA,
        ],
    )
    def gk(table_hbm, idx_hbm, out_hbm, idx_v, rows_v, sem_a, sem_b, sem_w):
        wid = lax.axis_index("s") * nc + lax.axis_index("c")
        pltpu.sync_copy(idx_hbm.at[wid], idx_v)
        copies = [
            pltpu.async_copy(
                table_hbm.at[idx_v.at[j]], rows_v.at[pl.ds(j * ch, ch)],
                sem_a if j < ga else sem_b,
            )
            for j in range(nchunk)
        ]
        for c in copies[:ga]:
            c.wait()
        wa = pltpu.async_copy(
            rows_v.at[pl.ds(0, ga * ch)],
            out_hbm.at[pl.ds(wid * bpw, ga * ch)], sem_w)
        for c in copies[ga:]:
            c.wait()
        wb = pltpu.async_copy(
            rows_v.at[pl.ds(ga * ch, bpw - ga * ch)],
            out_hbm.at[pl.ds(wid * bpw + ga * ch, bpw - ga * ch)], sem_w)
        wa.wait()
        wb.wait()

    return gk, nw, nchunk, ch, bpad


def kernel(prod_rule_idx_seq, edge_nodes, edge_symbols, node_symbols,
           atom_embed, bond_embed, W_layers, b_layers, W_out, b_out):
    table = _build_table(
        edge_symbols,
        node_symbols,
        edge_nodes.astype(jnp.int32),
        atom_embed,
        bond_embed,
        W_layers,
        b_layers,
        W_out,
        b_out.reshape(1, _D),
    )
    bsz, length = prod_rule_idx_seq.shape
    batch = bsz * length
    gk, nw, nchunk, ch, bpad = _gather_call(batch)
    bpw = batch // nw
    idx2 = prod_rule_idx_seq.reshape(nw, bpw).astype(jnp.int32)
    idx3 = jnp.pad(idx2, ((0, 0), (0, bpad - bpw))).reshape(nw, nchunk, ch)
    flat = gk(table, idx3)
    return flat.reshape(bsz, length, _D)


# speedup vs baseline: 1.3941x; 1.3941x over previous
import functools

import jax
import jax.numpy as jnp
from jax import lax
from jax.experimental import pallas as pl
from jax.experimental.pallas import tpu as pltpu
from jax.experimental.pallas import tpu_sc as plsc

_NUM_RULES = 1000
_NODES = 8
_EDGES = 12
_D = 64
_LAYERS = 3
_NSYM = 50
_PLANES = _EDGES + _NODES
_TABLE_ROWS = 1024


def _split3(x):
    f32 = jnp.float32
    hi = x.astype(jnp.bfloat16).astype(f32)
    r = x - hi
    mid = r.astype(jnp.bfloat16).astype(f32)
    return hi, mid, r - mid


def _table_body(esym, nsym, en, aemb, bemb, wl, bl, wot, bo, out,
                ohs, h, agg, tab):
    R = _NUM_RULES
    f32 = jnp.float32
    dn = (((1,), (0,)), ((), ()))
    dnt = (((1,), (1,)), ((), ()))

    tab[:, :] = jnp.zeros((2 * _D, _D), f32)
    tab[0:_NSYM, :] = aemb[:, :]
    tab[pl.ds(_D, _NSYM), :] = bemb[:, :]
    iota = lax.broadcasted_iota(jnp.int32, (R, _D), 1)

    def onehot_embed(nplanes, table):
        p1, p2, p3 = _split3(table)
        o = ohs[pl.ds(0, nplanes * R), :]
        return (lax.dot_general(o, p1, dn, preferred_element_type=f32)
                + lax.dot_general(o, p2, dn, preferred_element_type=f32)
                + lax.dot_general(o, p3, dn, preferred_element_type=f32))

    for e in range(_EDGES):
        ohs[pl.ds(e * R, R), :] = (esym[:, e:e + 1] == iota).astype(f32)
    h[pl.ds(0, _EDGES * R), :] = onehot_embed(_EDGES, tab[0:_D, :])
    for n in range(_NODES):
        ohs[pl.ds(n * R, R), :] = (nsym[:, n:n + 1] == iota).astype(f32)
    h[pl.ds(_EDGES * R, _NODES * R), :] = onehot_embed(_NODES, tab[pl.ds(_D, _D), :])

    def nplane(i):
        return pl.ds(pl.multiple_of((_EDGES + i) * R, 8), R)

    node0 = _EDGES * R
    ends = [(en[e, 0], en[e, 1]) for e in range(_EDGES)]
    v = None
    for l in range(_LAYERS):
        last = l == _LAYERS - 1
        if not last:
            for e in range(_EDGES):
                a, b = ends[e]
                agg[pl.ds(e * R, R), :] = (
                    h[pl.ds(e * R, R), :] + h[nplane(a), :] + h[nplane(b), :]
                )
        agg[pl.ds(node0, _NODES * R), :] = h[pl.ds(node0, _NODES * R), :]
        for e in range(_EDGES):
            for k in range(2):
                i = ends[e][k]
                agg[nplane(i), :] = agg[nplane(i), :] + h[pl.ds(e * R, R), :]
        if last:
            v = agg[pl.ds(node0 + (_NODES - 1) * R, R), :]
        else:
            h[:, :] = jnp.maximum(
                lax.dot_general(agg[:, :], wl[l], dnt,
                                preferred_element_type=f32) + bl[l:l + 1, :],
                0.0)
    out[0:R, :] = jnp.tanh(
        lax.dot_general(v, wot[:, :], dnt, preferred_element_type=f32) + bo[:, :])
    out[pl.ds(R, _TABLE_ROWS - R), :] = jnp.zeros((_TABLE_ROWS - R, _D), f32)


def _build_table(esym, nsym, en, aemb, bemb, wl, bl, wot, bo):
    R = _NUM_RULES
    vmem = pl.BlockSpec(memory_space=pltpu.VMEM)
    return pl.pallas_call(
        _table_body,
        out_shape=jax.ShapeDtypeStruct((_TABLE_ROWS, _D), jnp.float32),
        in_specs=[
            vmem,
            vmem,
            pl.BlockSpec(memory_space=pltpu.SMEM),
            vmem,
            vmem,
            vmem,
            vmem,
            vmem,
            vmem,
        ],
        out_specs=pl.BlockSpec(memory_space=pltpu.VMEM),
        scratch_shapes=[
            pltpu.VMEM((_EDGES * R, _D), jnp.float32),
            pltpu.VMEM((_PLANES * R, _D), jnp.float32),
            pltpu.VMEM((_PLANES * R, _D), jnp.float32),
            pltpu.VMEM((2 * _D, _D), jnp.float32),
        ],
    )(esym, nsym, en, aemb, bemb, wl, bl, wot, bo)


@functools.lru_cache(maxsize=None)
def _gather_call(batch):
    info = plsc.get_sparse_core_info()
    nc, ns = info.num_cores, info.num_subcores
    nw = nc * ns
    bpw = batch // nw
    ch = 64
    nchunk = bpw // ch
    mesh = plsc.VectorSubcoreMesh(core_axis_name="c", subcore_axis_name="s")

    @functools.partial(
        pl.kernel,
        mesh=mesh,
        compiler_params=pltpu.CompilerParams(use_tc_tiling_on_sc=False),
        out_type=jax.ShapeDtypeStruct((batch, _D), jnp.float32),
        scratch_types=[
            pltpu.VMEM((nchunk, ch), jnp.int32),
            pltpu.VMEM((bpw, _D), jnp.float32),
            pltpu.SemaphoreType.DMA,
            pltpu.SemaphoreType.DMA,
            pltpu.SemaphoreType.DMA,
        ],
    )
    def gk(table_hbm, idx_hbm, out_hbm, idx_v, rows_v, sem_a, sem_b, sem_w):
        wid = lax.axis_index("s") * nc + lax.axis_index("c")
        ga = nchunk // 2
        pltpu.sync_copy(idx_hbm.at[wid], idx_v)
        copies = [
            pltpu.async_copy(
                table_hbm.at[idx_v.at[j]], rows_v.at[pl.ds(j * ch, ch)],
                sem_a if j < ga else sem_b,
            )
            for j in range(nchunk)
        ]
        for c in copies[:ga]:
            c.wait()
        wa = pltpu.async_copy(
            rows_v.at[pl.ds(0, ga * ch)],
            out_hbm.at[pl.ds(wid * bpw, ga * ch)], sem_w)
        for c in copies[ga:]:
            c.wait()
        wb = pltpu.async_copy(
            rows_v.at[pl.ds(ga * ch, bpw - ga * ch)],
            out_hbm.at[pl.ds(wid * bpw + ga * ch, bpw - ga * ch)], sem_w)
        wa.wait()
        wb.wait()

    return gk, nw, nchunk, ch


def kernel(prod_rule_idx_seq, edge_nodes, edge_symbols, node_symbols,
           atom_embed, bond_embed, W_layers, b_layers, W_out, b_out):
    table = _build_table(
        edge_symbols,
        node_symbols,
        edge_nodes.astype(jnp.int32),
        atom_embed,
        bond_embed,
        W_layers,
        b_layers,
        W_out,
        b_out.reshape(1, _D),
    )
    bsz, length = prod_rule_idx_seq.shape
    batch = bsz * length
    gk, nw, nchunk, ch = _gather_call(batch)
    idx3 = prod_rule_idx_seq.reshape(nw, nchunk, ch).astype(jnp.int32)
    flat = gk(table, idx3)
    return flat.reshape(bsz, length, _D)


# speedup vs baseline: 1.4161x; 1.0158x over previous
import functools

import jax
import jax.numpy as jnp
from jax import lax
from jax.experimental import pallas as pl
from jax.experimental.pallas import tpu as pltpu
from jax.experimental.pallas import tpu_sc as plsc

_NUM_RULES = 1000
_NODES = 8
_EDGES = 12
_D = 64
_LAYERS = 3
_NSYM = 50
_PLANES = _EDGES + _NODES
_TABLE_ROWS = 1024


def _split3(x):
    f32 = jnp.float32
    hi = x.astype(jnp.bfloat16).astype(f32)
    r = x - hi
    mid = r.astype(jnp.bfloat16).astype(f32)
    return hi, mid, r - mid


def _table_body(esym, nsym, en, aemb, bemb, wl, bl, wot, bo, out,
                ohs, h, agg, tab):
    R = _NUM_RULES
    f32 = jnp.float32
    dn = (((1,), (0,)), ((), ()))
    dnt = (((1,), (1,)), ((), ()))

    tab[:, :] = jnp.zeros((2 * _D, _D), f32)
    tab[0:_NSYM, :] = aemb[:, :]
    tab[pl.ds(_D, _NSYM), :] = bemb[:, :]
    iota = lax.broadcasted_iota(jnp.int32, (R, _D), 1)

    def onehot_embed(nplanes, table):
        p1, p2, p3 = _split3(table)
        o = ohs[pl.ds(0, nplanes * R), :]
        return (lax.dot_general(o, p1, dn, preferred_element_type=f32)
                + lax.dot_general(o, p2, dn, preferred_element_type=f32)
                + lax.dot_general(o, p3, dn, preferred_element_type=f32))

    for e in range(_EDGES):
        ohs[pl.ds(e * R, R), :] = (esym[:, e:e + 1] == iota).astype(f32)
    h[pl.ds(0, _EDGES * R), :] = onehot_embed(_EDGES, tab[0:_D, :])
    for n in range(_NODES):
        ohs[pl.ds(n * R, R), :] = (nsym[:, n:n + 1] == iota).astype(f32)
    h[pl.ds(_EDGES * R, _NODES * R), :] = onehot_embed(_NODES, tab[pl.ds(_D, _D), :])

    def nplane(i):
        return pl.ds(pl.multiple_of((_EDGES + i) * R, 8), R)

    node0 = _EDGES * R
    ends = [(en[e, 0], en[e, 1]) for e in range(_EDGES)]
    v = None
    for l in range(_LAYERS):
        last = l == _LAYERS - 1
        if not last:
            for e in range(_EDGES):
                a, b = ends[e]
                agg[pl.ds(e * R, R), :] = (
                    h[pl.ds(e * R, R), :] + h[nplane(a), :] + h[nplane(b), :]
                )
        agg[pl.ds(node0, _NODES * R), :] = h[pl.ds(node0, _NODES * R), :]
        for e in range(_EDGES):
            for k in range(2):
                i = ends[e][k]
                agg[nplane(i), :] = agg[nplane(i), :] + h[pl.ds(e * R, R), :]
        if last:
            v = agg[pl.ds(node0 + (_NODES - 1) * R, R), :]
        else:
            h[:, :] = jnp.maximum(
                lax.dot_general(agg[:, :], wl[l], dnt,
                                preferred_element_type=f32) + bl[l:l + 1, :],
                0.0)
    out[0:R, :] = jnp.tanh(
        lax.dot_general(v, wot[:, :], dnt, preferred_element_type=f32) + bo[:, :])
    out[pl.ds(R, _TABLE_ROWS - R), :] = jnp.zeros((_TABLE_ROWS - R, _D), f32)


def _build_table(esym, nsym, en, aemb, bemb, wl, bl, wot, bo):
    R = _NUM_RULES
    vmem = pl.BlockSpec(memory_space=pltpu.VMEM)
    return pl.pallas_call(
        _table_body,
        out_shape=jax.ShapeDtypeStruct((_TABLE_ROWS, _D), jnp.float32),
        in_specs=[
            vmem,
            vmem,
            pl.BlockSpec(memory_space=pltpu.SMEM),
            vmem,
            vmem,
            vmem,
            vmem,
            vmem,
            vmem,
        ],
        out_specs=pl.BlockSpec(memory_space=pltpu.VMEM),
        scratch_shapes=[
            pltpu.VMEM((_EDGES * R, _D), jnp.float32),
            pltpu.VMEM((_PLANES * R, _D), jnp.float32),
            pltpu.VMEM((_PLANES * R, _D), jnp.float32),
            pltpu.VMEM((2 * _D, _D), jnp.float32),
        ],
    )(esym, nsym, en, aemb, bemb, wl, bl, wot, bo)


@functools.lru_cache(maxsize=None)
def _gather_call(batch):
    info = plsc.get_sparse_core_info()
    nc, ns = info.num_cores, info.num_subcores
    nw = nc * ns
    bpw = batch // nw
    ch = 64
    nchunk = bpw // ch
    mesh = plsc.VectorSubcoreMesh(core_axis_name="c", subcore_axis_name="s")

    @functools.partial(
        pl.kernel,
        mesh=mesh,
        compiler_params=pltpu.CompilerParams(use_tc_tiling_on_sc=False),
        out_type=jax.ShapeDtypeStruct((batch, _D), jnp.float32),
        scratch_types=[
            pltpu.VMEM((nchunk, ch), jnp.int32),
            pltpu.VMEM((bpw, _D), jnp.float32),
            pltpu.SemaphoreType.DMA,
        ],
    )
    def gk(table_hbm, idx_hbm, out_hbm, idx_v, rows_v, sem):
        wid = lax.axis_index("s") * nc + lax.axis_index("c")
        pltpu.sync_copy(idx_hbm.at[wid], idx_v)
        copies = [
            pltpu.async_copy(
                table_hbm.at[idx_v.at[j]], rows_v.at[pl.ds(j * ch, ch)], sem
            )
            for j in range(nchunk)
        ]
        for c in copies:
            c.wait()
        pltpu.sync_copy(rows_v, out_hbm.at[pl.ds(wid * bpw, bpw)])

    return gk, nw, nchunk, ch


def kernel(prod_rule_idx_seq, edge_nodes, edge_symbols, node_symbols,
           atom_embed, bond_embed, W_layers, b_layers, W_out, b_out):
    table = _build_table(
        edge_symbols,
        node_symbols,
        edge_nodes.astype(jnp.int32),
        atom_embed,
        bond_embed,
        W_layers,
        b_layers,
        W_out,
        b_out.reshape(1, _D),
    )
    bsz, length = prod_rule_idx_seq.shape
    batch = bsz * length
    gk, nw, nchunk, ch = _gather_call(batch)
    idx3 = prod_rule_idx_seq.reshape(nw, nchunk, ch).astype(jnp.int32)
    flat = gk(table, idx3)
    return flat.reshape(bsz, length, _D)
